# baseline (speedup 1.0000x reference)
import functools

import jax
import jax.numpy as jnp
from jax import lax
from jax.experimental import pallas as pl
from jax.experimental.pallas import tpu as pltpu
from jax.experimental.pallas import tpu_sc as plsc

N = 10000
E = 320000
F = 128
T = 3
NP = 10240
G = NP // 128
NC, NS = 2, 16
NW = NC * NS
K = 128
CH = 128
JC = 80
JC0 = 160
EPW = JC * CH
EP = NW * EPW
ROWS_PER_TILE = NP // NS
DPAD = 10200

@functools.lru_cache(maxsize=None)
def _mesh():
    return plsc.VectorSubcoreMesh(
        core_axis_name="c", subcore_axis_name="s",
        num_cores=NC, num_subcores=NS)


def _deg_body(dst3_hbm, zdeg_hbm, ones_hbm, out_hbm, dstv, onesv, acc):
    c = lax.axis_index("c")
    s = lax.axis_index("s")
    w = c * NS + s
    pltpu.sync_copy(zdeg_hbm, acc.at[pl.ds(s * ROWS_PER_TILE, ROWS_PER_TILE)])
    pltpu.sync_copy(ones_hbm, onesv)
    pltpu.sync_copy(dst3_hbm.at[w], dstv)
    plsc.subcore_barrier()

    def body(j, carry):
        pltpu.sync_copy(onesv, acc.at[dstv.at[j]], add=True)
        return carry

    lax.fori_loop(0, JC, body, 0)
    plsc.subcore_barrier()
    pltpu.sync_copy(acc.at[pl.ds(s * ROWS_PER_TILE, ROWS_PER_TILE)],
                    out_hbm.at[c, pl.ds(s * ROWS_PER_TILE, ROWS_PER_TILE)])


@functools.lru_cache(maxsize=None)
def _deg_kernel():
    return pl.kernel(
        _deg_body, mesh=_mesh(),
        out_type=jax.ShapeDtypeStruct((NC, NP), jnp.float32),
        scratch_types=[
            pltpu.VMEM((JC, CH), jnp.int32),
            pltpu.VMEM((CH,), jnp.float32),
            pltpu.VMEM_SHARED((NP,), jnp.float32),
        ])


def _scatter_run(y_hbm, idx3_hbm, jc, s, idxb0, idxb1, buf0, buf1, acc,
                 sem0, sem1, isem0, isem1):
    pltpu.make_async_copy(idx3_hbm.at[s, 0], idxb0, isem0).start()
    pltpu.make_async_copy(idx3_hbm.at[s, 1], idxb1, isem1).start()
    pltpu.make_async_copy(idx3_hbm.at[s, 0], idxb0, isem0).wait()
    pltpu.make_async_copy(y_hbm.at[idxb0.at[0]], buf0, sem0).start()

    def body(i, carry):
        j0 = 2 * i
        j1 = 2 * i + 1
        pltpu.make_async_copy(y_hbm.at[idxb0.at[0]], buf0, sem0).wait()
        pltpu.make_async_copy(idx3_hbm.at[s, j1], idxb1, isem1).wait()
        pltpu.make_async_copy(y_hbm.at[idxb1.at[0]], buf1, sem1).start()
        pltpu.sync_copy(buf0, acc.at[idxb0.at[1]], add=True)
        pltpu.make_async_copy(
            idx3_hbm.at[s, jnp.minimum(j0 + 2, jc - 1)], idxb0, isem0).start()
        pltpu.make_async_copy(y_hbm.at[idxb1.at[0]], buf1, sem1).wait()
        pltpu.make_async_copy(
            idx3_hbm.at[s, jnp.minimum(j0 + 2, jc - 1)], idxb0, isem0).wait()
        pltpu.make_async_copy(y_hbm.at[idxb0.at[0]], buf0, sem0).start()
        pltpu.sync_copy(buf1, acc.at[idxb1.at[1]], add=True)
        pltpu.make_async_copy(
            idx3_hbm.at[s, jnp.minimum(j1 + 2, jc - 1)], idxb1, isem1).start()
        return carry

    lax.fori_loop(0, jc // 2, body, 0)
    pltpu.make_async_copy(y_hbm.at[idxb0.at[0]], buf0, sem0).wait()
    pltpu.make_async_copy(idx3_hbm.at[s, jc - 1], idxb1, isem1).wait()


def _scatter_body(y_hbm, idx3a_hbm, zmain_hbm, out_hbm,
                  idxb0, idxb1, buf0, buf1, acc, sem0, sem1, isem0, isem1):
    c = lax.axis_index("c")
    s = lax.axis_index("s")

    @pl.when(c == 0)
    def _():
        pltpu.sync_copy(zmain_hbm,
                        acc.at[pl.ds(s * ROWS_PER_TILE, ROWS_PER_TILE)])
        plsc.subcore_barrier()
        _scatter_run(y_hbm, idx3a_hbm, JC0, s, idxb0, idxb1, buf0, buf1,
                     acc, sem0, sem1, isem0, isem1)
        plsc.subcore_barrier()
        pltpu.sync_copy(acc.at[pl.ds(s * ROWS_PER_TILE, ROWS_PER_TILE)],
                        out_hbm.at[pl.ds(s * ROWS_PER_TILE, ROWS_PER_TILE)])


@functools.lru_cache(maxsize=None)
def _scatter_kernel():
    return pl.kernel(
        _scatter_body, mesh=_mesh(),
        out_type=jax.ShapeDtypeStruct((NP, F), jnp.float32),
        scratch_types=[
            pltpu.VMEM((2, CH), jnp.int32),
            pltpu.VMEM((2, CH), jnp.int32),
            pltpu.VMEM((CH, F), jnp.float32),
            pltpu.VMEM((CH, F), jnp.float32),
            pltpu.VMEM_SHARED((NP, F), jnp.float32),
            pltpu.SemaphoreType.DMA,
            pltpu.SemaphoreType.DMA,
            pltpu.SemaphoreType.DMA,
            pltpu.SemaphoreType.DMA,
        ])


def _gathx_body(xflat_hbm, fidx_hbm, out_hbm, idxv, rows, sem):
    c = lax.axis_index("c")
    s = lax.axis_index("s")

    @pl.when(jnp.logical_and(c == 0, s < T))
    def _():
        pltpu.sync_copy(fidx_hbm, idxv)
        pltpu.async_copy(xflat_hbm.at[idxv.at[s]], rows, sem).wait()
        pltpu.sync_copy(rows, out_hbm.at[pl.ds(s * K, K)])


@functools.lru_cache(maxsize=None)
def _gathx_kernel():
    return pl.kernel(
        _gathx_body, mesh=_mesh(),
        out_type=jax.ShapeDtypeStruct((T * K, F), jnp.float32),
        scratch_types=[
            pltpu.VMEM((T, K), jnp.int32),
            pltpu.VMEM((K, F), jnp.float32),
            pltpu.SemaphoreType.DMA,
        ])


def _scores_body(x_ref, p_ref, s_ref):
    g = pl.program_id(1)
    ph = p_ref[...]
    lanes = lax.broadcasted_iota(jnp.int32, (1, 128), 1)
    for j in range(8):
        xb = x_ref[0, j]
        sr = lax.dot_general(ph, xb, (((1,), (1,)), ((), ())))
        ridx = (g * 8 + j) * 128 + lanes
        s_ref[0, j, 0] = jnp.where(ridx < N, sr, -jnp.inf)[0]


def _scores(x4, p2):
    return pl.pallas_call(
        _scores_body,
        grid=(T, G // 8),
        in_specs=[
            pl.BlockSpec((1, 8, 128, F), lambda t, g: (t, g, 0, 0)),
            pl.BlockSpec((1, F), lambda t, g: (0, 0)),
        ],
        out_specs=pl.BlockSpec((1, 8, 1, 128), lambda t, g: (t, g, 0, 0)),
        out_shape=jax.ShapeDtypeStruct((T, G, 1, 128), jnp.float32),
    )(x4, p2).reshape(T, G, 128)


def _topk_body(s_ref, fidx_ref, vals_ref):
    s0 = s_ref[...]
    gi = (lax.broadcasted_iota(jnp.int32, (T, G, 128), 1) * 128
          + lax.broadcasted_iota(jnp.int32, (T, G, 128), 2))
    lane = lax.broadcasted_iota(jnp.int32, (T, 128), 1)

    def body(k, carry):
        s, vals, idxs = carry
        m = jnp.max(jnp.max(s, axis=2, keepdims=True), axis=1, keepdims=True)
        cand = jnp.where(s == m, gi, jnp.int32(2**30))
        i = jnp.min(jnp.min(cand, axis=2, keepdims=True), axis=1,
                    keepdims=True)
        vals = jnp.where(lane == k, m[:, :, 0], vals)
        idxs = jnp.where(lane == k, i[:, :, 0], idxs)
        s = jnp.where(gi == i, -jnp.inf, s)
        return s, vals, idxs

    _, vals, idxs = lax.fori_loop(
        0, K, body,
        (s0, jnp.zeros((T, 128), jnp.float32), jnp.zeros((T, 128), jnp.int32)))
    vals_ref[...] = vals
    toff = lax.broadcasted_iota(jnp.int32, (T, 128), 0) * NP
    fidx_ref[...] = idxs + toff


def _topk(scores3):
    return pl.pallas_call(
        _topk_body,
        out_shape=[
            jax.ShapeDtypeStruct((T, 128), jnp.int32),
            jax.ShapeDtypeStruct((T, 128), jnp.float32),
        ],
    )(scores3)


def _gru_body(xg_ref, vals_ref, w0_ref, wih_ref, whh_ref, bih_ref, bhh_ref,
              wf_ref):
    W = w0_ref[...]
    wih = wih_ref[...]
    whh = whh_ref[...]
    bih = bih_ref[...]
    bhh = bhh_ref[...]
    rr = lax.broadcasted_iota(jnp.int32, (F, F), 0)
    cc = lax.broadcasted_iota(jnp.int32, (F, F), 1)
    eye = jnp.where(rr == cc, 1.0, 0.0)
    for t in range(T):
        tv = jnp.tanh(vals_ref[t])[None, :]
        diag = eye * tv
        xt = lax.dot_general(diag, xg_ref[t], (((1,), (0,)), ((), ())))
        gi = lax.dot_general(xt, wih, (((1,), (1,)), ((), ()))) + bih
        gh = lax.dot_general(W, whh, (((1,), (1,)), ((), ()))) + bhh
        r = jax.nn.sigmoid(gi[:, :F] + gh[:, :F])
        z = jax.nn.sigmoid(gi[:, F:2 * F] + gh[:, F:2 * F])
        n = jnp.tanh(gi[:, 2 * F:] + r * gh[:, 2 * F:])
        W = (1.0 - z) * n + z * W
    wf_ref[...] = W


def _gru(xg, vals, w0, wih, whh, bih2, bhh2):
    return pl.pallas_call(
        _gru_body,
        out_shape=jax.ShapeDtypeStruct((F, F), jnp.float32),
    )(xg, vals, w0, wih, whh, bih2, bhh2)


def _dinv128(dp_ref):
    d = dp_ref[0] + dp_ref[1] + 1.0
    return lax.rsqrt(d)


def _xw_body(x_ref, w_ref, dp_ref, y_ref):
    xw = lax.dot_general(x_ref[...], w_ref[...], (((1,), (0,)), ((), ())))
    y_ref[...] = _dinv128(dp_ref) * xw


def _xw(x2, wf, dparts):
    B = 1024
    return pl.pallas_call(
        _xw_body,
        grid=(NP // B,),
        in_specs=[
            pl.BlockSpec((B, F), lambda i: (i, 0)),
            pl.BlockSpec((F, F), lambda i: (0, 0)),
            pl.BlockSpec((NC, B, 1), lambda i: (0, i, 0)),
        ],
        out_specs=pl.BlockSpec((B, F), lambda i: (i, 0)),
        out_shape=jax.ShapeDtypeStruct((NP, F), jnp.float32),
    )(x2, wf, dparts)


def _fin_body(a_ref, y_ref, dp_ref, o_ref):
    o_ref[...] = _dinv128(dp_ref) * (a_ref[...] + y_ref[...])


def _fin(aparts, y, dparts):
    B = 1024
    return pl.pallas_call(
        _fin_body,
        grid=(NP // B,),
        in_specs=[
            pl.BlockSpec((B, F), lambda i: (i, 0)),
            pl.BlockSpec((B, F), lambda i: (i, 0)),
            pl.BlockSpec((NC, B, 1), lambda i: (0, i, 0)),
        ],
        out_specs=pl.BlockSpec((B, F), lambda i: (i, 0)),
        out_shape=jax.ShapeDtypeStruct((NP, F), jnp.float32),
    )(aparts, y, dparts)


def kernel(x_seq, edge_index, W_init, p, W_ih, W_hh, b_ih, b_hh):
    f32 = jnp.float32
    x_pad = jnp.pad(x_seq, ((0, 0), (0, NP - N), (0, 0)))
    x4 = x_pad.reshape(T, G, 128, F)
    xflat = x_pad.reshape(T * NP, F)

    pad_dst = N + (jnp.arange(EP - E, dtype=jnp.int32) % (NP - N))
    src_f = jnp.concatenate(
        [edge_index[0], jnp.zeros((EP - E,), jnp.int32)])
    dst_f = jnp.concatenate([edge_index[1], pad_dst])
    idx3a = jnp.stack([src_f.reshape(NS, JC0, CH),
                       dst_f.reshape(NS, JC0, CH)], axis=2)

    dst_p = dst_f.reshape(NW, JC, CH)
    zdeg = jnp.zeros((ROWS_PER_TILE,), f32)
    ones1 = jnp.ones((CH,), f32)
    zmain = jnp.zeros((ROWS_PER_TILE, F), f32)

    dparts = _deg_kernel()(dst_p, zdeg, ones1).reshape(NC, NP, 1)

    phat = p / (jnp.linalg.norm(p) + 1e-16)
    scores3 = _scores(x4, phat.reshape(1, F))
    fidx, vals = _topk(scores3)
    xg = _gathx_kernel()(xflat, fidx).reshape(T, K, F)
    wf = _gru(xg, vals, W_init, W_ih, W_hh,
              b_ih.reshape(1, 3 * F), b_hh.reshape(1, 3 * F))

    y = _xw(x_pad[2], wf, dparts)
    apart = _scatter_kernel()(y, idx3a, zmain)
    out = _fin(apart, y, dparts)
    return out[:N]


# speedup vs baseline: 2.6424x; 2.6424x over previous
import functools

import jax
import jax.numpy as jnp
from jax import lax
from jax.experimental import pallas as pl
from jax.experimental.pallas import tpu as pltpu
from jax.experimental.pallas import tpu_sc as plsc

N = 10000
E = 320000
F = 128
T = 3
NP = 10240
G = NP // 128
NC, NS = 2, 16
NW = NC * NS
K = 128
CH = 128
JC = 80
JC0 = 80
EPW = JC * CH
EP = NW * EPW
ROWS_PER_TILE = NP // NS
DPAD = 10200

@functools.lru_cache(maxsize=None)
def _mesh():
    return plsc.VectorSubcoreMesh(
        core_axis_name="c", subcore_axis_name="s",
        num_cores=NC, num_subcores=NS)


def _deg_body(dst3_hbm, zdeg_hbm, ones_hbm, out_hbm, dstv, onesv, acc):
    c = lax.axis_index("c")
    s = lax.axis_index("s")
    w = c * NS + s
    pltpu.sync_copy(zdeg_hbm, acc.at[pl.ds(s * ROWS_PER_TILE, ROWS_PER_TILE)])
    pltpu.sync_copy(ones_hbm, onesv)
    pltpu.sync_copy(dst3_hbm.at[w], dstv)
    plsc.subcore_barrier()

    def body(j, carry):
        pltpu.sync_copy(onesv, acc.at[dstv.at[j]], add=True)
        return carry

    lax.fori_loop(0, JC, body, 0)
    plsc.subcore_barrier()
    pltpu.sync_copy(acc.at[pl.ds(s * ROWS_PER_TILE, ROWS_PER_TILE)],
                    out_hbm.at[c, pl.ds(s * ROWS_PER_TILE, ROWS_PER_TILE)])


@functools.lru_cache(maxsize=None)
def _deg_kernel():
    return pl.kernel(
        _deg_body, mesh=_mesh(),
        out_type=jax.ShapeDtypeStruct((NC, NP), jnp.float32),
        scratch_types=[
            pltpu.VMEM((JC, CH), jnp.int32),
            pltpu.VMEM((CH,), jnp.float32),
            pltpu.VMEM_SHARED((NP,), jnp.float32),
        ])


def _scatter_run(y_hbm, idx3_hbm, jc, s, idxb0, idxb1, buf0, buf1, acc,
                 sem0, sem1, isem0, isem1):
    pltpu.make_async_copy(idx3_hbm.at[s, 0], idxb0, isem0).start()
    pltpu.make_async_copy(idx3_hbm.at[s, 1], idxb1, isem1).start()
    pltpu.make_async_copy(idx3_hbm.at[s, 0], idxb0, isem0).wait()
    pltpu.make_async_copy(y_hbm.at[idxb0.at[0]], buf0, sem0).start()

    def body(i, carry):
        j0 = 2 * i
        j1 = 2 * i + 1
        pltpu.make_async_copy(y_hbm.at[idxb0.at[0]], buf0, sem0).wait()
        pltpu.make_async_copy(idx3_hbm.at[s, j1], idxb1, isem1).wait()
        pltpu.make_async_copy(y_hbm.at[idxb1.at[0]], buf1, sem1).start()
        pltpu.sync_copy(buf0, acc.at[idxb0.at[1]], add=True)
        pltpu.make_async_copy(
            idx3_hbm.at[s, jnp.minimum(j0 + 2, jc - 1)], idxb0, isem0).start()
        pltpu.make_async_copy(y_hbm.at[idxb1.at[0]], buf1, sem1).wait()
        pltpu.make_async_copy(
            idx3_hbm.at[s, jnp.minimum(j0 + 2, jc - 1)], idxb0, isem0).wait()
        pltpu.make_async_copy(y_hbm.at[idxb0.at[0]], buf0, sem0).start()
        pltpu.sync_copy(buf1, acc.at[idxb1.at[1]], add=True)
        pltpu.make_async_copy(
            idx3_hbm.at[s, jnp.minimum(j1 + 2, jc - 1)], idxb1, isem1).start()
        return carry

    lax.fori_loop(0, jc // 2, body, 0)
    pltpu.make_async_copy(y_hbm.at[idxb0.at[0]], buf0, sem0).wait()
    pltpu.make_async_copy(idx3_hbm.at[s, jc - 1], idxb1, isem1).wait()


def _scatter_body(y_hbm, idx3a_hbm, zmain_hbm, out_hbm,
                  idxb0, idxb1, buf0, buf1, acc, sem0, sem1, isem0, isem1):
    c = lax.axis_index("c")
    s = lax.axis_index("s")
    w = c * NS + s
    pltpu.sync_copy(zmain_hbm, acc.at[pl.ds(s * ROWS_PER_TILE, ROWS_PER_TILE)])
    plsc.subcore_barrier()
    _scatter_run(y_hbm, idx3a_hbm, JC0, w, idxb0, idxb1, buf0, buf1,
                 acc, sem0, sem1, isem0, isem1)
    plsc.subcore_barrier()
    pltpu.sync_copy(acc.at[pl.ds(s * ROWS_PER_TILE, ROWS_PER_TILE)],
                    out_hbm.at[c, pl.ds(s * ROWS_PER_TILE, ROWS_PER_TILE)])


@functools.lru_cache(maxsize=None)
def _scatter_kernel():
    return pl.kernel(
        _scatter_body, mesh=_mesh(),
        out_type=jax.ShapeDtypeStruct((NC, NP, F), jnp.float32),
        scratch_types=[
            pltpu.VMEM((2, CH), jnp.int32),
            pltpu.VMEM((2, CH), jnp.int32),
            pltpu.VMEM((CH, F), jnp.float32),
            pltpu.VMEM((CH, F), jnp.float32),
            pltpu.VMEM_SHARED((NP, F), jnp.float32),
            pltpu.SemaphoreType.DMA,
            pltpu.SemaphoreType.DMA,
            pltpu.SemaphoreType.DMA,
            pltpu.SemaphoreType.DMA,
        ])


def _gathx_body(xflat_hbm, fidx_hbm, out_hbm, idxv, rows, sem):
    c = lax.axis_index("c")
    s = lax.axis_index("s")

    @pl.when(jnp.logical_and(c == 0, s < T))
    def _():
        pltpu.sync_copy(fidx_hbm, idxv)
        pltpu.async_copy(xflat_hbm.at[idxv.at[s]], rows, sem).wait()
        pltpu.sync_copy(rows, out_hbm.at[pl.ds(s * K, K)])


@functools.lru_cache(maxsize=None)
def _gathx_kernel():
    return pl.kernel(
        _gathx_body, mesh=_mesh(),
        out_type=jax.ShapeDtypeStruct((T * K, F), jnp.float32),
        scratch_types=[
            pltpu.VMEM((T, K), jnp.int32),
            pltpu.VMEM((K, F), jnp.float32),
            pltpu.SemaphoreType.DMA,
        ])


def _scores_body(x_ref, p_ref, s_ref):
    g = pl.program_id(1)
    ph = p_ref[...]
    lanes = lax.broadcasted_iota(jnp.int32, (1, 128), 1)
    for j in range(8):
        xb = x_ref[0, j]
        sr = lax.dot_general(ph, xb, (((1,), (1,)), ((), ())))
        ridx = (g * 8 + j) * 128 + lanes
        s_ref[0, j, 0] = jnp.where(ridx < N, sr, -jnp.inf)[0]


def _scores(x4, p2):
    return pl.pallas_call(
        _scores_body,
        grid=(T, G // 8),
        in_specs=[
            pl.BlockSpec((1, 8, 128, F), lambda t, g: (t, g, 0, 0)),
            pl.BlockSpec((1, F), lambda t, g: (0, 0)),
        ],
        out_specs=pl.BlockSpec((1, 8, 1, 128), lambda t, g: (t, g, 0, 0)),
        out_shape=jax.ShapeDtypeStruct((T, G, 1, 128), jnp.float32),
    )(x4, p2).reshape(T, G, 128)


def _topk_body(s_ref, fidx_ref, vals_ref):
    s0 = s_ref[...]
    gi = (lax.broadcasted_iota(jnp.int32, (T, G, 128), 1) * 128
          + lax.broadcasted_iota(jnp.int32, (T, G, 128), 2))
    lane = lax.broadcasted_iota(jnp.int32, (T, 128), 1)

    def body(k, carry):
        s, vals, idxs = carry
        m = jnp.max(jnp.max(s, axis=2, keepdims=True), axis=1, keepdims=True)
        cand = jnp.where(s == m, gi, jnp.int32(2**30))
        i = jnp.min(jnp.min(cand, axis=2, keepdims=True), axis=1,
                    keepdims=True)
        vals = jnp.where(lane == k, m[:, :, 0], vals)
        idxs = jnp.where(lane == k, i[:, :, 0], idxs)
        s = jnp.where(gi == i, -jnp.inf, s)
        return s, vals, idxs

    _, vals, idxs = lax.fori_loop(
        0, K, body,
        (s0, jnp.zeros((T, 128), jnp.float32), jnp.zeros((T, 128), jnp.int32)))
    vals_ref[...] = vals
    toff = lax.broadcasted_iota(jnp.int32, (T, 128), 0) * NP
    fidx_ref[...] = idxs + toff


def _topk(scores3):
    return pl.pallas_call(
        _topk_body,
        out_shape=[
            jax.ShapeDtypeStruct((T, 128), jnp.int32),
            jax.ShapeDtypeStruct((T, 128), jnp.float32),
        ],
    )(scores3)


def _gru_body(xg_ref, vals_ref, w0_ref, wih_ref, whh_ref, bih_ref, bhh_ref,
              wf_ref):
    W = w0_ref[...]
    wih = wih_ref[...]
    whh = whh_ref[...]
    bih = bih_ref[...]
    bhh = bhh_ref[...]
    rr = lax.broadcasted_iota(jnp.int32, (F, F), 0)
    cc = lax.broadcasted_iota(jnp.int32, (F, F), 1)
    eye = jnp.where(rr == cc, 1.0, 0.0)
    for t in range(T):
        tv = jnp.tanh(vals_ref[t])[None, :]
        diag = eye * tv
        xt = lax.dot_general(diag, xg_ref[t], (((1,), (0,)), ((), ())))
        gi = lax.dot_general(xt, wih, (((1,), (1,)), ((), ()))) + bih
        gh = lax.dot_general(W, whh, (((1,), (1,)), ((), ()))) + bhh
        r = jax.nn.sigmoid(gi[:, :F] + gh[:, :F])
        z = jax.nn.sigmoid(gi[:, F:2 * F] + gh[:, F:2 * F])
        n = jnp.tanh(gi[:, 2 * F:] + r * gh[:, 2 * F:])
        W = (1.0 - z) * n + z * W
    wf_ref[...] = W


def _gru(xg, vals, w0, wih, whh, bih2, bhh2):
    return pl.pallas_call(
        _gru_body,
        out_shape=jax.ShapeDtypeStruct((F, F), jnp.float32),
    )(xg, vals, w0, wih, whh, bih2, bhh2)


def _dinv128(dp_ref):
    d = dp_ref[0] + dp_ref[1] + 1.0
    return lax.rsqrt(d)


def _xw_body(x_ref, w_ref, dp_ref, y_ref):
    xw = lax.dot_general(x_ref[...], w_ref[...], (((1,), (0,)), ((), ())))
    y_ref[...] = _dinv128(dp_ref) * xw


def _xw(x2, wf, dparts):
    B = 1024
    return pl.pallas_call(
        _xw_body,
        grid=(NP // B,),
        in_specs=[
            pl.BlockSpec((B, F), lambda i: (i, 0)),
            pl.BlockSpec((F, F), lambda i: (0, 0)),
            pl.BlockSpec((NC, B, 1), lambda i: (0, i, 0)),
        ],
        out_specs=pl.BlockSpec((B, F), lambda i: (i, 0)),
        out_shape=jax.ShapeDtypeStruct((NP, F), jnp.float32),
    )(x2, wf, dparts)


def _fin_body(a_ref, y_ref, dp_ref, o_ref):
    o_ref[...] = _dinv128(dp_ref) * (a_ref[0] + a_ref[1] + y_ref[...])


def _fin(aparts, y, dparts):
    B = 1024
    return pl.pallas_call(
        _fin_body,
        grid=(NP // B,),
        in_specs=[
            pl.BlockSpec((NC, B, F), lambda i: (0, i, 0)),
            pl.BlockSpec((B, F), lambda i: (i, 0)),
            pl.BlockSpec((NC, B, 1), lambda i: (0, i, 0)),
        ],
        out_specs=pl.BlockSpec((B, F), lambda i: (i, 0)),
        out_shape=jax.ShapeDtypeStruct((NP, F), jnp.float32),
    )(aparts, y, dparts)


def kernel(x_seq, edge_index, W_init, p, W_ih, W_hh, b_ih, b_hh):
    f32 = jnp.float32
    x_pad = jnp.pad(x_seq, ((0, 0), (0, NP - N), (0, 0)))
    x4 = x_pad.reshape(T, G, 128, F)
    xflat = x_pad.reshape(T * NP, F)

    pad_iota = jnp.arange(EP - E, dtype=jnp.int32)
    pad_dst = N + (pad_iota % (NP - N))
    pad_src = pad_iota % N
    src_f = jnp.concatenate([edge_index[0], pad_src])
    dst_f = jnp.concatenate([edge_index[1], pad_dst])
    idx3a = jnp.stack([src_f.reshape(NW, JC0, CH),
                       dst_f.reshape(NW, JC0, CH)], axis=2)

    dst_p = dst_f.reshape(NW, JC, CH)
    zdeg = jnp.zeros((ROWS_PER_TILE,), f32)
    ones1 = jnp.ones((CH,), f32)
    zmain = jnp.zeros((ROWS_PER_TILE, F), f32)

    dparts = _deg_kernel()(dst_p, zdeg, ones1).reshape(NC, NP, 1)

    phat = p / (jnp.linalg.norm(p) + 1e-16)
    scores3 = _scores(x4, phat.reshape(1, F))
    fidx, vals = _topk(scores3)
    xg = _gathx_kernel()(xflat, fidx).reshape(T, K, F)
    wf = _gru(xg, vals, W_init, W_ih, W_hh,
              b_ih.reshape(1, 3 * F), b_hh.reshape(1, 3 * F))

    y = _xw(x_pad[2], wf, dparts)
    aparts = _scatter_kernel()(y, idx3a, zmain)
    out = _fin(aparts, y, dparts)
    return out[:N]


# speedup vs baseline: 2.8052x; 1.0616x over previous
import functools

import jax
import jax.numpy as jnp
from jax import lax
from jax.experimental import pallas as pl
from jax.experimental.pallas import tpu as pltpu
from jax.experimental.pallas import tpu_sc as plsc

N = 10000
E = 320000
F = 128
T = 3
NP = 10240
G = NP // 128
NC, NS = 2, 16
NW = NC * NS
K = 128
CH = 128
JC = 80
JC0 = 80
EPW = JC * CH
EP = NW * EPW
ROWS_PER_TILE = NP // NS
DPAD = 10200

@functools.lru_cache(maxsize=None)
def _mesh():
    return plsc.VectorSubcoreMesh(
        core_axis_name="c", subcore_axis_name="s",
        num_cores=NC, num_subcores=NS)


def _deg_body(dst3_hbm, zdeg_hbm, ones_hbm, out_hbm, dstv, onesv, acc):
    c = lax.axis_index("c")
    s = lax.axis_index("s")
    w = c * NS + s
    pltpu.sync_copy(zdeg_hbm, acc.at[pl.ds(s * ROWS_PER_TILE, ROWS_PER_TILE)])
    pltpu.sync_copy(ones_hbm, onesv)
    pltpu.sync_copy(dst3_hbm.at[w], dstv)
    plsc.subcore_barrier()

    def body(j, carry):
        pltpu.sync_copy(onesv, acc.at[dstv.at[j]], add=True)
        return carry

    lax.fori_loop(0, JC, body, 0)
    plsc.subcore_barrier()
    pltpu.sync_copy(acc.at[pl.ds(s * ROWS_PER_TILE, ROWS_PER_TILE)],
                    out_hbm.at[c, pl.ds(s * ROWS_PER_TILE, ROWS_PER_TILE)])


@functools.lru_cache(maxsize=None)
def _deg_kernel():
    return pl.kernel(
        _deg_body, mesh=_mesh(),
        out_type=jax.ShapeDtypeStruct((NC, NP), jnp.float32),
        scratch_types=[
            pltpu.VMEM((JC, CH), jnp.int32),
            pltpu.VMEM((CH,), jnp.float32),
            pltpu.VMEM_SHARED((NP,), jnp.float32),
        ])


def _scatter_run(y_hbm, idx3_hbm, jc, s, idxb0, idxb1, buf0, buf1, acc,
                 sem0, sem1, isem0, isem1):
    pltpu.make_async_copy(idx3_hbm.at[s, 0], idxb0, isem0).start()
    pltpu.make_async_copy(idx3_hbm.at[s, 1], idxb1, isem1).start()
    pltpu.make_async_copy(idx3_hbm.at[s, 0], idxb0, isem0).wait()
    pltpu.make_async_copy(y_hbm.at[idxb0.at[0]], buf0, sem0).start()

    def body(i, carry):
        j0 = 2 * i
        j1 = 2 * i + 1
        pltpu.make_async_copy(y_hbm.at[idxb0.at[0]], buf0, sem0).wait()
        pltpu.make_async_copy(idx3_hbm.at[s, j1], idxb1, isem1).wait()
        pltpu.make_async_copy(y_hbm.at[idxb1.at[0]], buf1, sem1).start()
        pltpu.sync_copy(buf0, acc.at[idxb0.at[1]], add=True)
        pltpu.make_async_copy(
            idx3_hbm.at[s, jnp.minimum(j0 + 2, jc - 1)], idxb0, isem0).start()
        pltpu.make_async_copy(y_hbm.at[idxb1.at[0]], buf1, sem1).wait()
        pltpu.make_async_copy(
            idx3_hbm.at[s, jnp.minimum(j0 + 2, jc - 1)], idxb0, isem0).wait()
        pltpu.make_async_copy(y_hbm.at[idxb0.at[0]], buf0, sem0).start()
        pltpu.sync_copy(buf1, acc.at[idxb1.at[1]], add=True)
        pltpu.make_async_copy(
            idx3_hbm.at[s, jnp.minimum(j1 + 2, jc - 1)], idxb1, isem1).start()
        return carry

    lax.fori_loop(0, jc // 2, body, 0)
    pltpu.make_async_copy(y_hbm.at[idxb0.at[0]], buf0, sem0).wait()
    pltpu.make_async_copy(idx3_hbm.at[s, jc - 1], idxb1, isem1).wait()


def _scatter_body(y_hbm, idx3a_hbm, zmain_hbm, out_hbm,
                  idxb0, idxb1, buf0, buf1, acc, sem0, sem1, isem0, isem1):
    c = lax.axis_index("c")
    s = lax.axis_index("s")
    w = c * NS + s
    pltpu.sync_copy(zmain_hbm, acc.at[pl.ds(s * ROWS_PER_TILE, ROWS_PER_TILE)])
    plsc.subcore_barrier()
    _scatter_run(y_hbm, idx3a_hbm, JC0, w, idxb0, idxb1, buf0, buf1,
                 acc, sem0, sem1, isem0, isem1)
    plsc.subcore_barrier()
    pltpu.sync_copy(acc.at[pl.ds(s * ROWS_PER_TILE, ROWS_PER_TILE)],
                    out_hbm.at[c, pl.ds(s * ROWS_PER_TILE, ROWS_PER_TILE)])


@functools.lru_cache(maxsize=None)
def _scatter_kernel():
    return pl.kernel(
        _scatter_body, mesh=_mesh(),
        out_type=jax.ShapeDtypeStruct((NC, NP, F), jnp.float32),
        scratch_types=[
            pltpu.VMEM((2, CH), jnp.int32),
            pltpu.VMEM((2, CH), jnp.int32),
            pltpu.VMEM((CH, F), jnp.float32),
            pltpu.VMEM((CH, F), jnp.float32),
            pltpu.VMEM_SHARED((NP, F), jnp.float32),
            pltpu.SemaphoreType.DMA,
            pltpu.SemaphoreType.DMA,
            pltpu.SemaphoreType.DMA,
            pltpu.SemaphoreType.DMA,
        ])


def _gathx_body(xflat_hbm, fidx_hbm, out_hbm, idxv, rows, sem):
    c = lax.axis_index("c")
    s = lax.axis_index("s")

    @pl.when(jnp.logical_and(c == 0, s < T))
    def _():
        pltpu.sync_copy(fidx_hbm, idxv)
        pltpu.async_copy(xflat_hbm.at[idxv.at[s]], rows, sem).wait()
        pltpu.sync_copy(rows, out_hbm.at[pl.ds(s * K, K)])


@functools.lru_cache(maxsize=None)
def _gathx_kernel():
    return pl.kernel(
        _gathx_body, mesh=_mesh(),
        out_type=jax.ShapeDtypeStruct((T * K, F), jnp.float32),
        scratch_types=[
            pltpu.VMEM((T, K), jnp.int32),
            pltpu.VMEM((K, F), jnp.float32),
            pltpu.SemaphoreType.DMA,
        ])


def _scores_body(x_ref, p_ref, s_ref):
    g = pl.program_id(1)
    ph = p_ref[...]
    lanes = lax.broadcasted_iota(jnp.int32, (1, 128), 1)
    for j in range(8):
        xb = x_ref[0, j * 128:(j + 1) * 128]
        sr = lax.dot_general(ph, xb, (((1,), (1,)), ((), ())))
        ridx = (g * 8 + j) * 128 + lanes
        s_ref[0, j, 0] = jnp.where(ridx < N, sr, -jnp.inf)[0]


def _scores(x3, p2):
    return pl.pallas_call(
        _scores_body,
        grid=(T, G // 8),
        in_specs=[
            pl.BlockSpec((1, 1024, F), lambda t, g: (t, g, 0)),
            pl.BlockSpec((1, F), lambda t, g: (0, 0)),
        ],
        out_specs=pl.BlockSpec((1, 8, 1, 128), lambda t, g: (t, g, 0, 0)),
        out_shape=jax.ShapeDtypeStruct((T, G, 1, 128), jnp.float32),
    )(x3, p2).reshape(T, G, 128)


def _topk_body(s_ref, fidx_ref, vals_ref):
    s0 = s_ref[...]
    gi = (lax.broadcasted_iota(jnp.int32, (T, G, 128), 1) * 128
          + lax.broadcasted_iota(jnp.int32, (T, G, 128), 2))
    lane = lax.broadcasted_iota(jnp.int32, (T, 128), 1)

    def body(k, carry):
        s, vals, idxs = carry
        m = jnp.max(jnp.max(s, axis=2, keepdims=True), axis=1, keepdims=True)
        cand = jnp.where(s == m, gi, jnp.int32(2**30))
        i = jnp.min(jnp.min(cand, axis=2, keepdims=True), axis=1,
                    keepdims=True)
        vals = jnp.where(lane == k, m[:, :, 0], vals)
        idxs = jnp.where(lane == k, i[:, :, 0], idxs)
        s = jnp.where(gi == i, -jnp.inf, s)
        return s, vals, idxs

    _, vals, idxs = lax.fori_loop(
        0, K, body,
        (s0, jnp.zeros((T, 128), jnp.float32), jnp.zeros((T, 128), jnp.int32)))
    vals_ref[...] = vals
    toff = lax.broadcasted_iota(jnp.int32, (T, 128), 0) * N
    fidx_ref[...] = idxs + toff


def _topk(scores3):
    return pl.pallas_call(
        _topk_body,
        out_shape=[
            jax.ShapeDtypeStruct((T, 128), jnp.int32),
            jax.ShapeDtypeStruct((T, 128), jnp.float32),
        ],
    )(scores3)


def _gru_body(xg_ref, vals_ref, w0_ref, wih_ref, whh_ref, bih_ref, bhh_ref,
              wf_ref):
    W = w0_ref[...]
    wih = wih_ref[...]
    whh = whh_ref[...]
    bih = bih_ref[...]
    bhh = bhh_ref[...]
    rr = lax.broadcasted_iota(jnp.int32, (F, F), 0)
    cc = lax.broadcasted_iota(jnp.int32, (F, F), 1)
    eye = jnp.where(rr == cc, 1.0, 0.0)
    for t in range(T):
        tv = jnp.tanh(vals_ref[t])[None, :]
        diag = eye * tv
        xt = lax.dot_general(diag, xg_ref[t], (((1,), (0,)), ((), ())))
        gi = lax.dot_general(xt, wih, (((1,), (1,)), ((), ()))) + bih
        gh = lax.dot_general(W, whh, (((1,), (1,)), ((), ()))) + bhh
        r = jax.nn.sigmoid(gi[:, :F] + gh[:, :F])
        z = jax.nn.sigmoid(gi[:, F:2 * F] + gh[:, F:2 * F])
        n = jnp.tanh(gi[:, 2 * F:] + r * gh[:, 2 * F:])
        W = (1.0 - z) * n + z * W
    wf_ref[...] = W


def _gru(xg, vals, w0, wih, whh, bih2, bhh2):
    return pl.pallas_call(
        _gru_body,
        out_shape=jax.ShapeDtypeStruct((F, F), jnp.float32),
    )(xg, vals, w0, wih, whh, bih2, bhh2)


def _dinv128(dp_ref):
    d = dp_ref[0] + dp_ref[1] + 1.0
    return lax.rsqrt(d)


def _xw_body(x_ref, w_ref, dp_ref, y_ref):
    xw = lax.dot_general(x_ref[...], w_ref[...], (((1,), (0,)), ((), ())))
    y_ref[...] = _dinv128(dp_ref) * xw


def _xw(x2, wf, dparts):
    B = 1024
    return pl.pallas_call(
        _xw_body,
        grid=(NP // B,),
        in_specs=[
            pl.BlockSpec((B, F), lambda i: (i, 0)),
            pl.BlockSpec((F, F), lambda i: (0, 0)),
            pl.BlockSpec((NC, B, 1), lambda i: (0, i, 0)),
        ],
        out_specs=pl.BlockSpec((B, F), lambda i: (i, 0)),
        out_shape=jax.ShapeDtypeStruct((NP, F), jnp.float32),
    )(x2, wf, dparts)


def _fin_body(a_ref, y_ref, dp_ref, o_ref):
    o_ref[...] = _dinv128(dp_ref) * (a_ref[0] + a_ref[1] + y_ref[...])


def _fin(aparts, y, dparts):
    B = 1000
    return pl.pallas_call(
        _fin_body,
        grid=(N // B,),
        in_specs=[
            pl.BlockSpec((NC, B, F), lambda i: (0, i, 0)),
            pl.BlockSpec((B, F), lambda i: (i, 0)),
            pl.BlockSpec((NC, B, 1), lambda i: (0, i, 0)),
        ],
        out_specs=pl.BlockSpec((B, F), lambda i: (i, 0)),
        out_shape=jax.ShapeDtypeStruct((N, F), jnp.float32),
    )(aparts, y, dparts)


def kernel(x_seq, edge_index, W_init, p, W_ih, W_hh, b_ih, b_hh):
    f32 = jnp.float32
    xflat = x_seq.reshape(T * N, F)

    pad_iota = jnp.arange(EP - E, dtype=jnp.int32)
    pad_dst = N + (pad_iota % (NP - N))
    pad_src = pad_iota % N
    src_f = jnp.concatenate([edge_index[0], pad_src])
    dst_f = jnp.concatenate([edge_index[1], pad_dst])
    idx3a = jnp.stack([src_f.reshape(NW, JC0, CH),
                       dst_f.reshape(NW, JC0, CH)], axis=2)

    dst_p = dst_f.reshape(NW, JC, CH)
    zdeg = jnp.zeros((ROWS_PER_TILE,), f32)
    ones1 = jnp.ones((CH,), f32)
    zmain = jnp.zeros((ROWS_PER_TILE, F), f32)

    dparts = _deg_kernel()(dst_p, zdeg, ones1).reshape(NC, NP, 1)

    phat = p / (jnp.linalg.norm(p) + 1e-16)
    scores3 = _scores(x_seq, phat.reshape(1, F))
    fidx, vals = _topk(scores3)
    xg = _gathx_kernel()(xflat, fidx).reshape(T, K, F)
    wf = _gru(xg, vals, W_init, W_ih, W_hh,
              b_ih.reshape(1, 3 * F), b_hh.reshape(1, 3 * F))

    y = _xw(x_seq[2], wf, dparts)
    aparts = _scatter_kernel()(y, idx3a, zmain)
    out = _fin(aparts, y, dparts)
    return out


# speedup vs baseline: 3.0213x; 1.0770x over previous
import functools

import jax
import jax.numpy as jnp
from jax import lax
from jax.experimental import pallas as pl
from jax.experimental.pallas import tpu as pltpu
from jax.experimental.pallas import tpu_sc as plsc

N = 10000
E = 320000
F = 128
T = 3
NP = 10240
G = NP // 128
NC, NS = 2, 16
NW = NC * NS
K = 128
CH = 64
JC = 160
EPW = JC * CH
EP = NW * EPW
ROWS_PER_TILE = NP // NS
DPAD = 10200

@functools.lru_cache(maxsize=None)
def _mesh():
    return plsc.VectorSubcoreMesh(
        core_axis_name="c", subcore_axis_name="s",
        num_cores=NC, num_subcores=NS)


def _deg_body(dst3_hbm, zdeg_hbm, ones_hbm, out_hbm, dstv, onesv, acc):
    c = lax.axis_index("c")
    s = lax.axis_index("s")
    w = c * NS + s
    pltpu.sync_copy(zdeg_hbm, acc.at[pl.ds(s * ROWS_PER_TILE, ROWS_PER_TILE)])
    pltpu.sync_copy(ones_hbm, onesv)
    pltpu.sync_copy(dst3_hbm.at[w], dstv)
    plsc.subcore_barrier()

    def body(j, carry):
        pltpu.sync_copy(onesv, acc.at[dstv.at[j]], add=True)
        return carry

    lax.fori_loop(0, JC, body, 0)
    plsc.subcore_barrier()
    pltpu.sync_copy(acc.at[pl.ds(s * ROWS_PER_TILE, ROWS_PER_TILE)],
                    out_hbm.at[c, pl.ds(s * ROWS_PER_TILE, ROWS_PER_TILE)])


@functools.lru_cache(maxsize=None)
def _deg_kernel():
    return pl.kernel(
        _deg_body, mesh=_mesh(),
        out_type=jax.ShapeDtypeStruct((NC, NP), jnp.float32),
        scratch_types=[
            pltpu.VMEM((JC, CH), jnp.int32),
            pltpu.VMEM((CH,), jnp.float32),
            pltpu.VMEM_SHARED((NP,), jnp.float32),
        ])


def _scatter_run(y_hbm, idx3_hbm, w, idxb, bufs, acc, gsems, isems):
    for k in range(7):
        pltpu.make_async_copy(idx3_hbm.at[w, k], idxb.at[k], isems[k]).start()
    for k in range(3):
        pltpu.make_async_copy(idx3_hbm.at[w, k], idxb.at[k], isems[k]).wait()
        pltpu.make_async_copy(y_hbm.at[idxb.at[k].at[0]], bufs[k],
                              gsems[k]).start()

    def body(i, carry):
        j8 = 8 * i
        for k in range(8):
            j = j8 + k
            b = k % 4
            m = k % 8
            bn = (k + 3) % 4
            mp = (k + 3) % 8
            mq = (k + 7) % 8
            pltpu.make_async_copy(y_hbm.at[idxb.at[mp].at[0]], bufs[b],
                                  gsems[b]).wait()
            pltpu.sync_copy(bufs[b], acc.at[idxb.at[m].at[1]], add=True)
            pltpu.make_async_copy(
                idx3_hbm.at[w, jnp.minimum(j + 7, JC - 1)], idxb.at[mq],
                isems[mq]).start()
            pltpu.make_async_copy(idx3_hbm.at[w, 0], idxb.at[mp],
                                  isems[mp]).wait()
            pltpu.make_async_copy(y_hbm.at[idxb.at[mp].at[0]], bufs[bn],
                                  gsems[bn]).start()
        return carry

    lax.fori_loop(0, JC // 8, body, 0)
    for k in range(3):
        pltpu.make_async_copy(y_hbm.at[idxb.at[0].at[0]], bufs[k],
                              gsems[k]).wait()
    for k in range(3, 7):
        pltpu.make_async_copy(idx3_hbm.at[w, 0], idxb.at[k], isems[k]).wait()


def _scatter_body(y_hbm, idx3a_hbm, zmain_hbm, out_hbm,
                  idxb, buf0, buf1, buf2, buf3, acc,
                  g0, g1, g2, g3, i0, i1, i2, i3, i4, i5, i6, i7):
    c = lax.axis_index("c")
    s = lax.axis_index("s")
    w = c * NS + s
    pltpu.sync_copy(zmain_hbm, acc.at[pl.ds(s * ROWS_PER_TILE, ROWS_PER_TILE)])
    plsc.subcore_barrier()
    _scatter_run(y_hbm, idx3a_hbm, w, idxb, [buf0, buf1, buf2, buf3], acc,
                 [g0, g1, g2, g3], [i0, i1, i2, i3, i4, i5, i6, i7])
    plsc.subcore_barrier()
    pltpu.sync_copy(acc.at[pl.ds(s * ROWS_PER_TILE, ROWS_PER_TILE)],
                    out_hbm.at[c, pl.ds(s * ROWS_PER_TILE, ROWS_PER_TILE)])


@functools.lru_cache(maxsize=None)
def _scatter_kernel():
    return pl.kernel(
        _scatter_body, mesh=_mesh(),
        out_type=jax.ShapeDtypeStruct((NC, NP, F), jnp.float32),
        scratch_types=(
            [pltpu.VMEM((8, 2, CH), jnp.int32)]
            + [pltpu.VMEM((CH, F), jnp.float32) for _ in range(4)]
            + [pltpu.VMEM_SHARED((NP, F), jnp.float32)]
            + [pltpu.SemaphoreType.DMA for _ in range(12)]
        ))


def _gathx_body(xflat_hbm, fidx_hbm, out_hbm, idxv, rows, sem):
    c = lax.axis_index("c")
    s = lax.axis_index("s")

    @pl.when(jnp.logical_and(c == 0, s < T))
    def _():
        pltpu.sync_copy(fidx_hbm, idxv)
        pltpu.async_copy(xflat_hbm.at[idxv.at[s]], rows, sem).wait()
        pltpu.sync_copy(rows, out_hbm.at[pl.ds(s * K, K)])


@functools.lru_cache(maxsize=None)
def _gathx_kernel():
    return pl.kernel(
        _gathx_body, mesh=_mesh(),
        out_type=jax.ShapeDtypeStruct((T * K, F), jnp.float32),
        scratch_types=[
            pltpu.VMEM((T, K), jnp.int32),
            pltpu.VMEM((K, F), jnp.float32),
            pltpu.SemaphoreType.DMA,
        ])


def _scores_body(x_ref, p_ref, s_ref):
    g = pl.program_id(1)
    ph = p_ref[...]
    lanes = lax.broadcasted_iota(jnp.int32, (1, 128), 1)
    for j in range(8):
        xb = x_ref[0, j * 128:(j + 1) * 128]
        sr = lax.dot_general(ph, xb, (((1,), (1,)), ((), ())))
        ridx = (g * 8 + j) * 128 + lanes
        s_ref[0, j, 0] = jnp.where(ridx < N, sr, -jnp.inf)[0]


def _scores(x3, p2):
    return pl.pallas_call(
        _scores_body,
        grid=(T, G // 8),
        in_specs=[
            pl.BlockSpec((1, 1024, F), lambda t, g: (t, g, 0)),
            pl.BlockSpec((1, F), lambda t, g: (0, 0)),
        ],
        out_specs=pl.BlockSpec((1, 8, 1, 128), lambda t, g: (t, g, 0, 0)),
        out_shape=jax.ShapeDtypeStruct((T, G, 1, 128), jnp.float32),
    )(x3, p2).reshape(T, G, 128)


def _topk_body(s_ref, fidx_ref, vals_ref):
    s0 = s_ref[...]
    gi = (lax.broadcasted_iota(jnp.int32, (T, G, 128), 1) * 128
          + lax.broadcasted_iota(jnp.int32, (T, G, 128), 2))
    lane = lax.broadcasted_iota(jnp.int32, (T, 128), 1)

    def body(k, carry):
        s, vals, idxs = carry
        m = jnp.max(jnp.max(s, axis=2, keepdims=True), axis=1, keepdims=True)
        cand = jnp.where(s == m, gi, jnp.int32(2**30))
        i = jnp.min(jnp.min(cand, axis=2, keepdims=True), axis=1,
                    keepdims=True)
        vals = jnp.where(lane == k, m[:, :, 0], vals)
        idxs = jnp.where(lane == k, i[:, :, 0], idxs)
        s = jnp.where(gi == i, -jnp.inf, s)
        return s, vals, idxs

    _, vals, idxs = lax.fori_loop(
        0, K, body,
        (s0, jnp.zeros((T, 128), jnp.float32), jnp.zeros((T, 128), jnp.int32)))
    vals_ref[...] = vals
    toff = lax.broadcasted_iota(jnp.int32, (T, 128), 0) * N
    fidx_ref[...] = idxs + toff


def _topk(scores3):
    return pl.pallas_call(
        _topk_body,
        out_shape=[
            jax.ShapeDtypeStruct((T, 128), jnp.int32),
            jax.ShapeDtypeStruct((T, 128), jnp.float32),
        ],
    )(scores3)


def _gru_body(xg_ref, vals_ref, w0_ref, wih_ref, whh_ref, bih_ref, bhh_ref,
              wf_ref):
    W = w0_ref[...]
    wih = wih_ref[...]
    whh = whh_ref[...]
    bih = bih_ref[...]
    bhh = bhh_ref[...]
    rr = lax.broadcasted_iota(jnp.int32, (F, F), 0)
    cc = lax.broadcasted_iota(jnp.int32, (F, F), 1)
    eye = jnp.where(rr == cc, 1.0, 0.0)
    for t in range(T):
        tv = jnp.tanh(vals_ref[t])[None, :]
        diag = eye * tv
        xt = lax.dot_general(diag, xg_ref[t], (((1,), (0,)), ((), ())))
        gi = lax.dot_general(xt, wih, (((1,), (1,)), ((), ()))) + bih
        gh = lax.dot_general(W, whh, (((1,), (1,)), ((), ()))) + bhh
        r = jax.nn.sigmoid(gi[:, :F] + gh[:, :F])
        z = jax.nn.sigmoid(gi[:, F:2 * F] + gh[:, F:2 * F])
        n = jnp.tanh(gi[:, 2 * F:] + r * gh[:, 2 * F:])
        W = (1.0 - z) * n + z * W
    wf_ref[...] = W


def _gru(xg, vals, w0, wih, whh, bih2, bhh2):
    return pl.pallas_call(
        _gru_body,
        out_shape=jax.ShapeDtypeStruct((F, F), jnp.float32),
    )(xg, vals, w0, wih, whh, bih2, bhh2)


def _dinv128(dp_ref):
    d = dp_ref[0] + dp_ref[1] + 1.0
    return lax.rsqrt(d)


def _xw_body(x_ref, w_ref, dp_ref, y_ref):
    xw = lax.dot_general(x_ref[...], w_ref[...], (((1,), (0,)), ((), ())))
    y_ref[...] = _dinv128(dp_ref) * xw


def _xw(x2, wf, dparts):
    B = 1024
    return pl.pallas_call(
        _xw_body,
        grid=(NP // B,),
        in_specs=[
            pl.BlockSpec((B, F), lambda i: (i, 0)),
            pl.BlockSpec((F, F), lambda i: (0, 0)),
            pl.BlockSpec((NC, B, 1), lambda i: (0, i, 0)),
        ],
        out_specs=pl.BlockSpec((B, F), lambda i: (i, 0)),
        out_shape=jax.ShapeDtypeStruct((NP, F), jnp.float32),
    )(x2, wf, dparts)


def _fin_body(a_ref, y_ref, dp_ref, o_ref):
    o_ref[...] = _dinv128(dp_ref) * (a_ref[0] + a_ref[1] + y_ref[...])


def _fin(aparts, y, dparts):
    B = 1000
    return pl.pallas_call(
        _fin_body,
        grid=(N // B,),
        in_specs=[
            pl.BlockSpec((NC, B, F), lambda i: (0, i, 0)),
            pl.BlockSpec((B, F), lambda i: (i, 0)),
            pl.BlockSpec((NC, B, 1), lambda i: (0, i, 0)),
        ],
        out_specs=pl.BlockSpec((B, F), lambda i: (i, 0)),
        out_shape=jax.ShapeDtypeStruct((N, F), jnp.float32),
    )(aparts, y, dparts)


def kernel(x_seq, edge_index, W_init, p, W_ih, W_hh, b_ih, b_hh):
    f32 = jnp.float32
    xflat = x_seq.reshape(T * N, F)

    pad_iota = jnp.arange(EP - E, dtype=jnp.int32)
    pad_dst = N + (pad_iota % (NP - N))
    pad_src = pad_iota % N
    src_f = jnp.concatenate([edge_index[0], pad_src])
    dst_f = jnp.concatenate([edge_index[1], pad_dst])
    idx3a = jnp.stack([src_f.reshape(NW, JC, CH),
                       dst_f.reshape(NW, JC, CH)], axis=2)

    dst_p = dst_f.reshape(NW, JC, CH)
    zdeg = jnp.zeros((ROWS_PER_TILE,), f32)
    ones1 = jnp.ones((CH,), f32)
    zmain = jnp.zeros((ROWS_PER_TILE, F), f32)

    dparts = _deg_kernel()(dst_p, zdeg, ones1).reshape(NC, NP, 1)

    phat = p / (jnp.linalg.norm(p) + 1e-16)
    scores3 = _scores(x_seq, phat.reshape(1, F))
    fidx, vals = _topk(scores3)
    xg = _gathx_kernel()(xflat, fidx).reshape(T, K, F)
    wf = _gru(xg, vals, W_init, W_ih, W_hh,
              b_ih.reshape(1, 3 * F), b_hh.reshape(1, 3 * F))

    y = _xw(x_seq[2], wf, dparts)
    aparts = _scatter_kernel()(y, idx3a, zmain)
    out = _fin(aparts, y, dparts)
    return out
